# baseline (speedup 1.0000x reference)
import functools

import jax
import jax.numpy as jnp
from jax import lax
from jax.experimental import pallas as pl
from jax.experimental.pallas import tpu as pltpu
from jax.experimental.pallas import tpu_sc as plsc

_N = 50000
_D = 128
_L = 3
_R = 3
_E = 500000
_NP = 50176
_EP = 524288
_EB = 128
_K = 4
_NC, _NS = 2, 16
_NW = _NC * _NS
_ROWS_T = _NP // _NS
_ZB = 392
_CH = 16
_NCH = _D // _CH
_BLK_T = _EP // _NW // _EB
_HB = _BLK_T // 2
_STEPS = _HB // _K
_DC = 8
_BN = 512


def _sc_mesh():
    return plsc.VectorSubcoreMesh(
        core_axis_name="c", subcore_axis_name="s", num_cores=_NC, num_subcores=_NS
    )


_SC_PARAMS = dict(
    compiler_params=pltpu.CompilerParams(use_tc_tiling_on_sc=False),
)


def _sc_degrees(edges, onehot, zeros):
    @functools.partial(
        pl.kernel,
        out_type=jax.ShapeDtypeStruct((_NC * _NP, _DC), jnp.float32),
        mesh=_sc_mesh(),
        scratch_types=[
            pltpu.VMEM_SHARED((_NP, _DC), jnp.float32),
            pltpu.VMEM((_EB, _DC), jnp.float32),
            pltpu.VMEM((_BLK_T, _EB), jnp.int32),
        ],
        **_SC_PARAMS,
    )
    def k(e_hbm, oh_hbm, z_hbm, out_hbm, accum, ohbuf, ebuf):
        c = lax.axis_index("c")
        s = lax.axis_index("s")
        w = s * _NC + c

        pltpu.sync_copy(z_hbm, accum.at[pl.ds(s * _ROWS_T, _ROWS_T)])
        plsc.subcore_barrier()

        for r in range(_R):
            for which in range(2):
                pltpu.sync_copy(oh_hbm.at[2 * r + which], ohbuf)
                pltpu.sync_copy(e_hbm.at[r, which, w], ebuf)

                def body(j, _):
                    pltpu.sync_copy(ohbuf, accum.at[ebuf.at[j]], add=True)
                    return 0

                lax.fori_loop(0, _BLK_T, body, 0)
        plsc.subcore_barrier()
        pltpu.sync_copy(
            accum.at[pl.ds(s * _ROWS_T, _ROWS_T)],
            out_hbm.at[pl.ds(c * _NP + s * _ROWS_T, _ROWS_T)],
        )

    return k(edges, onehot, zeros)


def _sc_msgpass(y, edges):
    @functools.partial(
        pl.kernel,
        out_type=jax.ShapeDtypeStruct((_R, _NCH, _NC * _NP, _CH), jnp.float32),
        mesh=_sc_mesh(),
        scratch_types=[
            pltpu.VMEM_SHARED((_NP, _CH), jnp.float32),
            pltpu.VMEM((_ZB, _CH), jnp.float32),
            pltpu.VMEM((_HB, _EB), jnp.int32),
            pltpu.VMEM((_HB, _EB), jnp.int32),
            [pltpu.VMEM((_EB, _CH), jnp.float32) for _ in range(_K)],
            pltpu.SemaphoreType.DMA,
            pltpu.SemaphoreType.DMA,
        ],
        **_SC_PARAMS,
    )
    def k(y_hbm, e_hbm, out_hbm, accum, zbuf, sidx, didx, rows, gsem, ssem):
        c = lax.axis_index("c")
        s = lax.axis_index("s")
        w = s * _NC + c

        def zrow(i, _):
            zbuf[i, pl.ds(0, 16)] = jnp.zeros((16,), jnp.float32)
            return 0

        lax.fori_loop(0, _ZB, zrow, 0)

        for r in range(_R):
            for ch in range(_NCH):
                for zi in range(8):
                    pltpu.sync_copy(
                        zbuf, accum.at[pl.ds(s * _ROWS_T + zi * _ZB, _ZB)]
                    )
                plsc.subcore_barrier()

                for half in range(2):
                    pltpu.sync_copy(
                        e_hbm.at[r, 0, w, pl.ds(half * _HB, _HB)], sidx
                    )
                    pltpu.sync_copy(
                        e_hbm.at[r, 1, w, pl.ds(half * _HB, _HB)], didx
                    )

                    def step(j, _):
                        @pl.when(j > 0)
                        def _():
                            for b in range(_K):
                                pltpu.make_async_copy(
                                    rows[b], accum.at[didx.at[0]], ssem
                                ).wait()

                        for b in range(_K):
                            pltpu.async_copy(
                                y_hbm.at[r, ch].at[sidx.at[j * _K + b]],
                                rows[b],
                                gsem,
                            )
                        for b in range(_K):
                            pltpu.make_async_copy(
                                y_hbm.at[r, ch].at[sidx.at[0]], rows[b], gsem
                            ).wait()
                            pltpu.async_copy(
                                rows[b],
                                accum.at[didx.at[j * _K + b]],
                                ssem,
                                add=True,
                            )
                        return 0

                    lax.fori_loop(0, _STEPS, step, 0)
                    for b in range(_K):
                        pltpu.make_async_copy(
                            rows[b], accum.at[didx.at[0]], ssem
                        ).wait()
                plsc.subcore_barrier()
                pltpu.sync_copy(
                    accum.at[pl.ds(s * _ROWS_T, _ROWS_T)],
                    out_hbm.at[r, ch, pl.ds(c * _NP + s * _ROWS_T, _ROWS_T)],
                )
                plsc.subcore_barrier()

    return k(y, edges)


def _deg_scales(dp0, dp1, r):
    deg = dp0 + dp1
    dout = lax.rsqrt(jnp.maximum(deg[:, 2 * r : 2 * r + 1], 1.0))
    din = lax.rsqrt(jnp.maximum(deg[:, 2 * r + 1 : 2 * r + 2], 1.0))
    return dout, din


def _write_chunks(y_ref, r, y):
    for ch in range(_NCH):
        y_ref[r, ch] = y[:, ch * _CH : (ch + 1) * _CH]


def _tc_layer0(x_pad, degp, wrel, fcw):
    def body(x_ref, dp_ref, wr_ref, fw_ref, y_ref):
        xb = x_ref[...]
        fw = fw_ref[...]
        for r in range(_R):
            dout, _ = _deg_scales(dp_ref[0], dp_ref[1], r)
            t = xb * dout
            y = jnp.dot(
                jnp.dot(t, wr_ref[r], preferred_element_type=jnp.float32),
                fw,
                preferred_element_type=jnp.float32,
            )
            _write_chunks(y_ref, r, y)

    return pl.pallas_call(
        body,
        grid=(_NP // _BN,),
        in_specs=[
            pl.BlockSpec((_BN, _D), lambda i: (i, 0)),
            pl.BlockSpec((2, _BN, _DC), lambda i: (0, i, 0)),
            pl.BlockSpec((_R, _D, _D), lambda i: (0, 0, 0)),
            pl.BlockSpec((_D, _D), lambda i: (0, 0)),
        ],
        out_specs=pl.BlockSpec((_R, _NCH, _BN, _CH), lambda i: (0, 0, i, 0)),
        out_shape=jax.ShapeDtypeStruct((_R, _NCH, _NP, _CH), jnp.float32),
    )(x_pad, degp, wrel, fcw)


def _assemble_h(s_ref, dp_ref, bprev_ref, fwprev_ref, fbprev_ref):
    brow = (
        jnp.dot(
            (bprev_ref[0] + bprev_ref[1] + bprev_ref[2]).reshape(1, _D),
            fwprev_ref[...],
            preferred_element_type=jnp.float32,
        )
        + fbprev_ref[...].reshape(1, _D)
    )
    h = jnp.zeros((_BN, _D), jnp.float32)
    for r in range(_R):
        _, din = _deg_scales(dp_ref[0], dp_ref[1], r)
        sr = jnp.concatenate(
            [s_ref[r, ch, 0] + s_ref[r, ch, 1] for ch in range(_NCH)], axis=1
        )
        h = h + sr * din
    return h + brow


def _tc_layer(s, degp, wrel, fcw, bprev, fwprev, fbprev, g, b, m, v):
    def body(
        s_ref, dp_ref, wr_ref, fw_ref, bp_ref, fwp_ref, fbp_ref,
        g_ref, b_ref, m_ref, v_ref, y_ref,
    ):
        i = pl.program_id(0)
        h = _assemble_h(s_ref, dp_ref, bp_ref, fwp_ref, fbp_ref)
        h = (h - m_ref[...].reshape(1, _D)) * lax.rsqrt(
            v_ref[...].reshape(1, _D) + 1e-5
        ) * g_ref[...].reshape(1, _D) + b_ref[...].reshape(1, _D)
        h = jnp.maximum(h, 0.0)
        rowid = i * _BN + lax.broadcasted_iota(jnp.int32, (_BN, 1), 0)
        h = jnp.where(rowid < _N, h, 0.0)
        fw = fw_ref[...]
        for r in range(_R):
            dout, _ = _deg_scales(dp_ref[0], dp_ref[1], r)
            y = jnp.dot(
                jnp.dot(h * dout, wr_ref[r], preferred_element_type=jnp.float32),
                fw,
                preferred_element_type=jnp.float32,
            )
            _write_chunks(y_ref, r, y)

    vec = lambda: pl.BlockSpec((_D,), lambda i: (0,))
    return pl.pallas_call(
        body,
        grid=(_NP // _BN,),
        in_specs=[
            pl.BlockSpec((_R, _NCH, 2, _BN, _CH), lambda i: (0, 0, 0, i, 0)),
            pl.BlockSpec((2, _BN, _DC), lambda i: (0, i, 0)),
            pl.BlockSpec((_R, _D, _D), lambda i: (0, 0, 0)),
            pl.BlockSpec((_D, _D), lambda i: (0, 0)),
            pl.BlockSpec((_R, _D), lambda i: (0, 0)),
            pl.BlockSpec((_D, _D), lambda i: (0, 0)),
            vec(), vec(), vec(), vec(), vec(),
        ],
        out_specs=pl.BlockSpec((_R, _NCH, _BN, _CH), lambda i: (0, 0, i, 0)),
        out_shape=jax.ShapeDtypeStruct((_R, _NCH, _NP, _CH), jnp.float32),
    )(s, degp, wrel, fcw, bprev, fwprev, fbprev, g, b, m, v)


def _tc_final(s, degp, bprev, fwprev, fbprev):
    def body(s_ref, dp_ref, bp_ref, fwp_ref, fbp_ref, o_ref):
        o_ref[...] = _assemble_h(s_ref, dp_ref, bp_ref, fwp_ref, fbp_ref)

    return pl.pallas_call(
        body,
        grid=(_NP // _BN,),
        in_specs=[
            pl.BlockSpec((_R, _NCH, 2, _BN, _CH), lambda i: (0, 0, 0, i, 0)),
            pl.BlockSpec((2, _BN, _DC), lambda i: (0, i, 0)),
            pl.BlockSpec((_R, _D), lambda i: (0, 0)),
            pl.BlockSpec((_D, _D), lambda i: (0, 0)),
            pl.BlockSpec((_D,), lambda i: (0,)),
        ],
        out_specs=pl.BlockSpec((_BN, _D), lambda i: (i, 0)),
        out_shape=jax.ShapeDtypeStruct((_NP, _D), jnp.float32),
    )(s, degp, bprev, fwprev, fbprev)


@jax.jit
def _run(x, edge_seq, edge_knn, edge_dis, Wrel, brel, fcW, fcb,
         bn_gamma, bn_beta, bn_mean, bn_var):
    x_pad = jnp.pad(x, ((0, _NP - _N), (0, 0)))
    pad = jnp.full((2, _EP - _E), _N, jnp.int32)
    edges = jnp.stack(
        [jnp.concatenate([e, pad], axis=1) for e in (edge_seq, edge_knn, edge_dis)]
    ).reshape(_R, 2, _NW, _BLK_T, _EB)
    onehot = jnp.broadcast_to(
        jnp.eye(_DC, dtype=jnp.float32)[:6, None, :], (6, _EB, _DC)
    )
    zeros = jnp.zeros((_ROWS_T, _DC), jnp.float32)

    degp = _sc_degrees(edges, onehot, zeros).reshape(_NC, _NP, _DC)

    def body(l, s):
        def stage0(_s):
            return _tc_layer0(x_pad, degp, Wrel[0], fcW[0])

        def stagen(s):
            lm = l - 1
            idx = lambda a, i: lax.dynamic_index_in_dim(a, i, 0, keepdims=False)
            return _tc_layer(
                s.reshape(_R, _NCH, _NC, _NP, _CH), degp,
                idx(Wrel, l), idx(fcW, l), idx(brel, lm), idx(fcW, lm),
                idx(fcb, lm), idx(bn_gamma, lm), idx(bn_beta, lm),
                idx(bn_mean, lm), idx(bn_var, lm),
            )

        y = lax.cond(l == 0, stage0, stagen, s)
        return _sc_msgpass(y, edges)

    s0 = jnp.zeros((_R, _NCH, _NC * _NP, _CH), jnp.float32)
    s = lax.fori_loop(0, _L, body, s0)
    out = _tc_final(
        s.reshape(_R, _NCH, _NC, _NP, _CH), degp, brel[2], fcW[2], fcb[2]
    )
    return out[:_N]


def kernel(x, edge_seq, edge_knn, edge_dis, Wrel, brel, fcW, fcb,
           bn_gamma, bn_beta, bn_mean, bn_var):
    return _run(x, edge_seq, edge_knn, edge_dis, Wrel, brel, fcW, fcb,
                bn_gamma, bn_beta, bn_mean, bn_var)


# speedup vs baseline: 1.0397x; 1.0397x over previous
import functools

import jax
import jax.numpy as jnp
from jax import lax
from jax.experimental import pallas as pl
from jax.experimental.pallas import tpu as pltpu
from jax.experimental.pallas import tpu_sc as plsc

_N = 50000
_D = 128
_L = 3
_R = 3
_E = 500000
_NP = 50176
_EP = 524288
_EB = 128
_K = 8
_NC, _NS = 2, 16
_NW = _NC * _NS
_ROWS_T = _NP // _NS
_ZB = 392
_CH = 16
_NCH = _D // _CH
_BLK_T = _EP // _NW // _EB
_HB = _BLK_T // 2
_STEPS = _HB // _K
_DC = 8
_BN = 512


def _sc_mesh():
    return plsc.VectorSubcoreMesh(
        core_axis_name="c", subcore_axis_name="s", num_cores=_NC, num_subcores=_NS
    )


_SC_PARAMS = dict(
    compiler_params=pltpu.CompilerParams(use_tc_tiling_on_sc=False),
)


def _sc_degrees(edges, onehot, zeros):
    @functools.partial(
        pl.kernel,
        out_type=jax.ShapeDtypeStruct((_NC * _NP, _DC), jnp.float32),
        mesh=_sc_mesh(),
        scratch_types=[
            pltpu.VMEM_SHARED((_NP, _DC), jnp.float32),
            pltpu.VMEM((_EB, _DC), jnp.float32),
            pltpu.VMEM((_BLK_T, _EB), jnp.int32),
        ],
        **_SC_PARAMS,
    )
    def k(e_hbm, oh_hbm, z_hbm, out_hbm, accum, ohbuf, ebuf):
        c = lax.axis_index("c")
        s = lax.axis_index("s")
        w = s * _NC + c

        pltpu.sync_copy(z_hbm, accum.at[pl.ds(s * _ROWS_T, _ROWS_T)])
        plsc.subcore_barrier()

        for r in range(_R):
            for which in range(2):
                pltpu.sync_copy(oh_hbm.at[2 * r + which], ohbuf)
                pltpu.sync_copy(e_hbm.at[r, which, w], ebuf)

                def body(j, _):
                    pltpu.sync_copy(ohbuf, accum.at[ebuf.at[j]], add=True)
                    return 0

                lax.fori_loop(0, _BLK_T, body, 0)
        plsc.subcore_barrier()
        pltpu.sync_copy(
            accum.at[pl.ds(s * _ROWS_T, _ROWS_T)],
            out_hbm.at[pl.ds(c * _NP + s * _ROWS_T, _ROWS_T)],
        )

    return k(edges, onehot, zeros)


def _sc_msgpass(y, edges):
    @functools.partial(
        pl.kernel,
        out_type=jax.ShapeDtypeStruct((_R, _NCH, _NC * _NP, _CH), jnp.float32),
        mesh=_sc_mesh(),
        scratch_types=[
            pltpu.VMEM_SHARED((_NP, _CH), jnp.float32),
            pltpu.VMEM((_ZB, _CH), jnp.float32),
            pltpu.VMEM((_HB, _EB), jnp.int32),
            pltpu.VMEM((_HB, _EB), jnp.int32),
            [pltpu.VMEM((_EB, _CH), jnp.float32) for _ in range(_K)],
            pltpu.SemaphoreType.DMA,
            pltpu.SemaphoreType.DMA,
        ],
        **_SC_PARAMS,
    )
    def k(y_hbm, e_hbm, out_hbm, accum, zbuf, sidx, didx, rows, gsem, ssem):
        c = lax.axis_index("c")
        s = lax.axis_index("s")
        w = s * _NC + c

        def zrow(i, _):
            zbuf[i, pl.ds(0, 16)] = jnp.zeros((16,), jnp.float32)
            return 0

        lax.fori_loop(0, _ZB, zrow, 0)

        for r in range(_R):

            def chround(ch, _):
                for zi in range(8):
                    pltpu.sync_copy(
                        zbuf, accum.at[pl.ds(s * _ROWS_T + zi * _ZB, _ZB)]
                    )
                plsc.subcore_barrier()

                for half in range(2):
                    pltpu.sync_copy(
                        e_hbm.at[r, 0, w, pl.ds(half * _HB, _HB)], sidx
                    )
                    pltpu.sync_copy(
                        e_hbm.at[r, 1, w, pl.ds(half * _HB, _HB)], didx
                    )

                    def step(j, _):
                        @pl.when(j > 0)
                        def _():
                            for b in range(_K):
                                pltpu.make_async_copy(
                                    rows[b], accum.at[didx.at[0]], ssem
                                ).wait()

                        for b in range(_K):
                            pltpu.async_copy(
                                y_hbm.at[r, ch].at[sidx.at[j * _K + b]],
                                rows[b],
                                gsem,
                            )
                        for b in range(_K):
                            pltpu.make_async_copy(
                                y_hbm.at[r, ch].at[sidx.at[0]], rows[b], gsem
                            ).wait()
                            pltpu.async_copy(
                                rows[b],
                                accum.at[didx.at[j * _K + b]],
                                ssem,
                                add=True,
                            )
                        return 0

                    lax.fori_loop(0, _STEPS, step, 0)
                    for b in range(_K):
                        pltpu.make_async_copy(
                            rows[b], accum.at[didx.at[0]], ssem
                        ).wait()
                plsc.subcore_barrier()
                pltpu.sync_copy(
                    accum.at[pl.ds(s * _ROWS_T, _ROWS_T)],
                    out_hbm.at[r, ch, pl.ds(c * _NP + s * _ROWS_T, _ROWS_T)],
                )
                plsc.subcore_barrier()
                return 0

            lax.fori_loop(0, _NCH, chround, 0)

    return k(y, edges)


def _deg_scales(dp0, dp1, r):
    deg = dp0 + dp1
    dout = lax.rsqrt(jnp.maximum(deg[:, 2 * r : 2 * r + 1], 1.0))
    din = lax.rsqrt(jnp.maximum(deg[:, 2 * r + 1 : 2 * r + 2], 1.0))
    return dout, din


def _write_chunks(y_ref, r, y):
    for ch in range(_NCH):
        y_ref[r, ch] = y[:, ch * _CH : (ch + 1) * _CH]


def _tc_layer0(x_pad, degp, wrel, fcw):
    def body(x_ref, dp_ref, wr_ref, fw_ref, y_ref):
        xb = x_ref[...]
        fw = fw_ref[...]
        for r in range(_R):
            dout, _ = _deg_scales(dp_ref[0], dp_ref[1], r)
            t = xb * dout
            y = jnp.dot(
                jnp.dot(t, wr_ref[r], preferred_element_type=jnp.float32),
                fw,
                preferred_element_type=jnp.float32,
            )
            _write_chunks(y_ref, r, y)

    return pl.pallas_call(
        body,
        grid=(_NP // _BN,),
        in_specs=[
            pl.BlockSpec((_BN, _D), lambda i: (i, 0)),
            pl.BlockSpec((2, _BN, _DC), lambda i: (0, i, 0)),
            pl.BlockSpec((_R, _D, _D), lambda i: (0, 0, 0)),
            pl.BlockSpec((_D, _D), lambda i: (0, 0)),
        ],
        out_specs=pl.BlockSpec((_R, _NCH, _BN, _CH), lambda i: (0, 0, i, 0)),
        out_shape=jax.ShapeDtypeStruct((_R, _NCH, _NP, _CH), jnp.float32),
    )(x_pad, degp, wrel, fcw)


def _assemble_h(s_ref, dp_ref, bprev_ref, fwprev_ref, fbprev_ref):
    brow = (
        jnp.dot(
            (bprev_ref[0] + bprev_ref[1] + bprev_ref[2]).reshape(1, _D),
            fwprev_ref[...],
            preferred_element_type=jnp.float32,
        )
        + fbprev_ref[...].reshape(1, _D)
    )
    h = jnp.zeros((_BN, _D), jnp.float32)
    for r in range(_R):
        _, din = _deg_scales(dp_ref[0], dp_ref[1], r)
        sr = jnp.concatenate(
            [s_ref[r, ch, 0] + s_ref[r, ch, 1] for ch in range(_NCH)], axis=1
        )
        h = h + sr * din
    return h + brow


def _tc_layer(s, degp, wrel, fcw, bprev, fwprev, fbprev, g, b, m, v):
    def body(
        s_ref, dp_ref, wr_ref, fw_ref, bp_ref, fwp_ref, fbp_ref,
        g_ref, b_ref, m_ref, v_ref, y_ref,
    ):
        i = pl.program_id(0)
        h = _assemble_h(s_ref, dp_ref, bp_ref, fwp_ref, fbp_ref)
        h = (h - m_ref[...].reshape(1, _D)) * lax.rsqrt(
            v_ref[...].reshape(1, _D) + 1e-5
        ) * g_ref[...].reshape(1, _D) + b_ref[...].reshape(1, _D)
        h = jnp.maximum(h, 0.0)
        rowid = i * _BN + lax.broadcasted_iota(jnp.int32, (_BN, 1), 0)
        h = jnp.where(rowid < _N, h, 0.0)
        fw = fw_ref[...]
        for r in range(_R):
            dout, _ = _deg_scales(dp_ref[0], dp_ref[1], r)
            y = jnp.dot(
                jnp.dot(h * dout, wr_ref[r], preferred_element_type=jnp.float32),
                fw,
                preferred_element_type=jnp.float32,
            )
            _write_chunks(y_ref, r, y)

    vec = lambda: pl.BlockSpec((_D,), lambda i: (0,))
    return pl.pallas_call(
        body,
        grid=(_NP // _BN,),
        in_specs=[
            pl.BlockSpec((_R, _NCH, 2, _BN, _CH), lambda i: (0, 0, 0, i, 0)),
            pl.BlockSpec((2, _BN, _DC), lambda i: (0, i, 0)),
            pl.BlockSpec((_R, _D, _D), lambda i: (0, 0, 0)),
            pl.BlockSpec((_D, _D), lambda i: (0, 0)),
            pl.BlockSpec((_R, _D), lambda i: (0, 0)),
            pl.BlockSpec((_D, _D), lambda i: (0, 0)),
            vec(), vec(), vec(), vec(), vec(),
        ],
        out_specs=pl.BlockSpec((_R, _NCH, _BN, _CH), lambda i: (0, 0, i, 0)),
        out_shape=jax.ShapeDtypeStruct((_R, _NCH, _NP, _CH), jnp.float32),
    )(s, degp, wrel, fcw, bprev, fwprev, fbprev, g, b, m, v)


def _tc_final(s, degp, bprev, fwprev, fbprev):
    def body(s_ref, dp_ref, bp_ref, fwp_ref, fbp_ref, o_ref):
        o_ref[...] = _assemble_h(s_ref, dp_ref, bp_ref, fwp_ref, fbp_ref)

    return pl.pallas_call(
        body,
        grid=(_NP // _BN,),
        in_specs=[
            pl.BlockSpec((_R, _NCH, 2, _BN, _CH), lambda i: (0, 0, 0, i, 0)),
            pl.BlockSpec((2, _BN, _DC), lambda i: (0, i, 0)),
            pl.BlockSpec((_R, _D), lambda i: (0, 0)),
            pl.BlockSpec((_D, _D), lambda i: (0, 0)),
            pl.BlockSpec((_D,), lambda i: (0,)),
        ],
        out_specs=pl.BlockSpec((_BN, _D), lambda i: (i, 0)),
        out_shape=jax.ShapeDtypeStruct((_NP, _D), jnp.float32),
    )(s, degp, bprev, fwprev, fbprev)


@jax.jit
def _run(x, edge_seq, edge_knn, edge_dis, Wrel, brel, fcW, fcb,
         bn_gamma, bn_beta, bn_mean, bn_var):
    x_pad = jnp.pad(x, ((0, _NP - _N), (0, 0)))
    pad = jnp.full((2, _EP - _E), _N, jnp.int32)
    edges = jnp.stack(
        [jnp.concatenate([e, pad], axis=1) for e in (edge_seq, edge_knn, edge_dis)]
    ).reshape(_R, 2, _NW, _BLK_T, _EB)
    onehot = jnp.broadcast_to(
        jnp.eye(_DC, dtype=jnp.float32)[:6, None, :], (6, _EB, _DC)
    )
    zeros = jnp.zeros((_ROWS_T, _DC), jnp.float32)

    degp = _sc_degrees(edges, onehot, zeros).reshape(_NC, _NP, _DC)

    def body(l, s):
        def stage0(_s):
            return _tc_layer0(x_pad, degp, Wrel[0], fcW[0])

        def stagen(s):
            lm = l - 1
            idx = lambda a, i: lax.dynamic_index_in_dim(a, i, 0, keepdims=False)
            return _tc_layer(
                s.reshape(_R, _NCH, _NC, _NP, _CH), degp,
                idx(Wrel, l), idx(fcW, l), idx(brel, lm), idx(fcW, lm),
                idx(fcb, lm), idx(bn_gamma, lm), idx(bn_beta, lm),
                idx(bn_mean, lm), idx(bn_var, lm),
            )

        y = lax.cond(l == 0, stage0, stagen, s)
        return _sc_msgpass(y, edges)

    s0 = jnp.zeros((_R, _NCH, _NC * _NP, _CH), jnp.float32)
    s = lax.fori_loop(0, _L, body, s0)
    out = _tc_final(
        s.reshape(_R, _NCH, _NC, _NP, _CH), degp, brel[2], fcW[2], fcb[2]
    )
    return out[:_N]


def kernel(x, edge_seq, edge_knn, edge_dis, Wrel, brel, fcW, fcb,
           bn_gamma, bn_beta, bn_mean, bn_var):
    return _run(x, edge_seq, edge_knn, edge_dis, Wrel, brel, fcW, fcb,
                bn_gamma, bn_beta, bn_mean, bn_var)
